# Initial kernel scaffold; baseline (speedup 1.0000x reference)
import jax
import jax.numpy as jnp
from jax.experimental import pallas as pl


def kernel(x, edge_index, W1, b1):
    raise NotImplementedError("write your pallas kernel here")


# speedup vs baseline: 26.7231x; 26.7231x over previous
import functools

import jax
import jax.numpy as jnp
from jax import lax
from jax.experimental import pallas as pl
from jax.experimental.pallas import tpu as pltpu
from jax.experimental.pallas import tpu_sc as plsc

_N = 10000
_E = 320000
_D = 128
_SCALE = 1.8

_NC = 2
_NS = 16
_NW = _NC * _NS
_EPT = _E // _NW
_CH = 80
_NCH = _EPT // _CH

_NPAD = 10240
_DEGW = 16

_ROWS_PT = _NPAD // _NS
_ZCH = 128


def _deg_body(dst2, zeros_hbm, deg_out, dst_v, hist_v, blk_v, hist_sh):
    c = lax.axis_index("c")
    s = lax.axis_index("s")
    wid = c * _NS + s
    pltpu.sync_copy(zeros_hbm, hist_v)
    pltpu.sync_copy(dst2.at[wid], dst_v)
    ones = jnp.full((16,), 1.0, jnp.float32)

    def step(i, carry):
        ix = dst_v[pl.ds(i * 16, 16)]
        plsc.addupdate_scatter(hist_v, [ix], ones)
        return carry

    lax.fori_loop(0, _EPT // 16, step, 0)
    pltpu.sync_copy(hist_v, hist_sh.at[s])
    plsc.subcore_barrier()
    sl = pl.ds(s * _ROWS_PT, _ROWS_PT)
    for k in range(_NS):
        pltpu.sync_copy(hist_sh.at[k, sl], blk_v.at[k])

    def red(i, carry):
        acc = blk_v[0, pl.ds(i * 16, 16)]
        for k in range(1, _NS):
            acc = acc + blk_v[k, pl.ds(i * 16, 16)]
        hist_v[pl.ds(i * 16, 16)] = acc
        return carry

    lax.fori_loop(0, _ROWS_PT // 16, red, 0)
    pltpu.sync_copy(hist_v.at[pl.ds(0, _ROWS_PT)], deg_out.at[c, sl])


def _deg_hist(dst2, zeros_npad):
    mesh = plsc.VectorSubcoreMesh(core_axis_name="c", subcore_axis_name="s", num_cores=_NC, num_subcores=_NS)
    return pl.kernel(
        _deg_body,
        out_type=jax.ShapeDtypeStruct((_NC, _NPAD), jnp.float32),
        mesh=mesh,
        compiler_params=pltpu.CompilerParams(needs_layout_passes=False),
        scratch_types=[
            pltpu.VMEM((_EPT,), jnp.int32),
            pltpu.VMEM((_NPAD,), jnp.float32),
            pltpu.VMEM((_NS, _ROWS_PT), jnp.float32),
            pltpu.VMEM_SHARED((_NS, _NPAD), jnp.float32),
        ],
    )(dst2, zeros_npad)


def _linear_body(x_ref, w_ref, b_ref, deg_ref, g_ref):
    h = lax.dot_general(
        x_ref[...], w_ref[...], (((1,), (1,)), ((), ())),
        preferred_element_type=jnp.float32,
    ) + b_ref[...]
    nrm = jnp.sqrt(jnp.sum(h * h, axis=1, keepdims=True))
    hn = h * (_SCALE / jnp.maximum(nrm, 1e-12))
    d = deg_ref[0, :_N] + deg_ref[1, :_N]
    dinv = lax.rsqrt(jnp.maximum(d + 1.0, 1.0))
    g_ref[...] = hn * dinv


def _linear(x, W1, b1, deg):
    return pl.pallas_call(
        _linear_body,
        out_shape=jax.ShapeDtypeStruct((_N, _D), jnp.float32),
    )(x, W1, b1.reshape(1, _D), deg)


def _prop_body(g_hbm, src3, dst3, zeros_hbm, s_out, src_v, dst_v, rows_v, s_sh, sem):
    c = lax.axis_index("c")
    s = lax.axis_index("s")
    wid = c * _NS + s
    for k in range(_ROWS_PT // _ZCH):
        pltpu.sync_copy(
            zeros_hbm, s_sh.at[pl.ds(s * _ROWS_PT + k * _ZCH, _ZCH)])
    pltpu.sync_copy(src3.at[wid], src_v)
    pltpu.sync_copy(dst3.at[wid], dst_v)
    plsc.subcore_barrier()

    def chunk(j, carry):
        pltpu.async_copy(g_hbm.at[src_v.at[j]], rows_v, sem).wait()
        pltpu.sync_copy(rows_v, s_sh.at[dst_v.at[j]], add=True)
        return carry

    lax.fori_loop(0, _NCH, chunk, 0)
    plsc.subcore_barrier()
    sl = pl.ds(s * _ROWS_PT, _ROWS_PT)
    pltpu.sync_copy(s_sh.at[sl], s_out.at[c, sl])


def _propagate(g, src3, dst3, zrows):
    mesh = plsc.VectorSubcoreMesh(core_axis_name="c", subcore_axis_name="s", num_cores=_NC, num_subcores=_NS)
    return pl.kernel(
        _prop_body,
        out_type=jax.ShapeDtypeStruct((_NC, _NPAD, _D), jnp.float32),
        mesh=mesh,
        scratch_types=[
            pltpu.VMEM((_NCH, _CH), jnp.int32),
            pltpu.VMEM((_NCH, _CH), jnp.int32),
            pltpu.VMEM((_CH, _D), jnp.float32),
            pltpu.VMEM_SHARED((_NPAD, _D), jnp.float32),
            pltpu.SemaphoreType.DMA,
        ],
    )(g, src3, dst3, zrows)


def _combine_body(g_ref, s_ref, deg_ref, o_ref):
    d = deg_ref[0, :_N] + deg_ref[1, :_N]
    dinv = lax.rsqrt(jnp.maximum(d + 1.0, 1.0))
    o_ref[...] = dinv * (g_ref[...] + s_ref[0, :_N] + s_ref[1, :_N])


def _combine(g, s_part, deg):
    return pl.pallas_call(
        _combine_body,
        out_shape=jax.ShapeDtypeStruct((_N, _D), jnp.float32),
    )(g, s_part, deg)


def kernel(x, edge_index, W1, b1):
    src3 = edge_index[0].reshape(_NW, _NCH, _CH)
    dst3 = edge_index[1].reshape(_NW, _NCH, _CH)
    dst2 = edge_index[1].reshape(_NW, _EPT)
    zeros_npad = jnp.zeros((_NPAD,), jnp.float32)
    zrows = jnp.zeros((_ZCH, _D), jnp.float32)

    deg = _deg_hist(dst2, zeros_npad).reshape(_NC, _NPAD, 1)
    g = _linear(x, W1, b1, deg)
    s_part = _propagate(g, src3, dst3, zrows)
    return _combine(g, s_part, deg)
